# baseline (speedup 1.0000x reference)
import jax
import jax.numpy as jnp
from jax import lax
from jax.experimental import pallas as pl
from jax.experimental.pallas import tpu as pltpu
from jax.experimental.pallas import tpu_sc as plsc

N = 10000
E = 320000
H = 128
G = 64

NP = 10240
NC = 2
NS = 16
NW = NC * NS
EPW = E // NW
CH = 128
GCH = 80
EPWP = GCH * CH
NPT = NP // NS
BR = 512
NB = NP // BR
LI = 4
KR = 2

_F32 = jnp.float32
_HIGH = lax.Precision.HIGHEST


def _sc_agg(h, src4, dst4, zrows, w):
    mesh = plsc.VectorSubcoreMesh(core_axis_name="c", subcore_axis_name="s")

    def body(h_hbm, src_hbm, dst_hbm, z_hbm, out_hbm, srcv, dstv, rows0,
             rows1, accum, g0, g1, d0, d1, d2, d3):
        rowsb = (rows0, rows1)
        gsems = (g0, g1)
        dsems = (d0, d1, d2, d3)
        cid = lax.axis_index("c")
        sid = lax.axis_index("s")
        wid = cid * NS + sid
        r0 = sid * NPT
        pltpu.sync_copy(z_hbm, accum.at[pl.ds(r0, NPT)])
        pltpu.sync_copy(src_hbm.at[wid], srcv)
        for j in range(LI):
            pltpu.async_copy(dst_hbm.at[wid, j], dstv.at[j], dsems[j])
        for k in range(KR):
            pltpu.async_copy(h_hbm.at[srcv.at[k, 0]], rowsb[k], gsems[k])
        plsc.subcore_barrier()

        def step(so, carry):
            for u in range(LI):
                g = so * LI + u
                k = u % KR
                pltpu.make_async_copy(h_hbm.at[srcv.at[u, 0]], rowsb[k],
                                      gsems[k]).wait()
                pltpu.make_async_copy(dst_hbm.at[wid, 0], dstv.at[u],
                                      dsems[u]).wait()
                pltpu.sync_copy(rowsb[k], accum.at[dstv.at[u, 0]], add=True)
                gn = jnp.minimum(g + LI, GCH - 1)
                pltpu.async_copy(dst_hbm.at[wid, gn], dstv.at[u], dsems[u])
                g2 = jnp.minimum(g + 2, GCH - 1)
                pltpu.async_copy(h_hbm.at[srcv.at[g2, 0]], rowsb[k], gsems[k])
            return carry

        lax.fori_loop(0, GCH // LI, step, 0)
        for k in range(KR):
            pltpu.make_async_copy(h_hbm.at[srcv.at[k, 0]], rowsb[k],
                                  gsems[k]).wait()
        for j in range(LI):
            pltpu.make_async_copy(dst_hbm.at[wid, 0], dstv.at[j], dsems[j]).wait()
        plsc.subcore_barrier()
        pltpu.sync_copy(accum.at[pl.ds(r0, NPT)],
                        out_hbm.at[cid, pl.ds(r0, NPT)])

    call = pl.kernel(
        body,
        out_type=jax.ShapeDtypeStruct((NC, NP, w), _F32),
        mesh=mesh,
        scratch_types=[
            pltpu.VMEM((GCH, 1, CH), jnp.int32),
            pltpu.VMEM((LI, 1, CH), jnp.int32),
            pltpu.VMEM((CH, w), _F32),
            pltpu.VMEM((CH, w), _F32),
            pltpu.VMEM_SHARED((NP, w), _F32),
            pltpu.SemaphoreType.DMA,
            pltpu.SemaphoreType.DMA,
            pltpu.SemaphoreType.DMA,
            pltpu.SemaphoreType.DMA,
            pltpu.SemaphoreType.DMA,
            pltpu.SemaphoreType.DMA,
        ],
    )
    return call(h, src4, dst4, zrows)


def _tc_layer0(s0, x_aug, wl, bl, wr):
    def body(s_ref, x_ref, wl_ref, bl_ref, wr_ref, h_ref, cnt_ref):
        s = s_ref[0] + s_ref[1]
        cnt = jnp.maximum(s[:, 5:6], 1.0)
        mean = s / cnt
        h = (jnp.dot(mean, wl_ref[...], preferred_element_type=_F32, precision=_HIGH)
             + bl_ref[...]
             + jnp.dot(x_ref[...], wr_ref[...], preferred_element_type=_F32, precision=_HIGH))
        h_ref[...] = jnp.maximum(h, 0.0)
        cnt_ref[...] = cnt

    return pl.pallas_call(
        body,
        grid=(NB,),
        in_specs=[
            pl.BlockSpec((NC, BR, H), lambda i: (0, i, 0)),
            pl.BlockSpec((BR, H), lambda i: (i, 0)),
            pl.BlockSpec((H, H), lambda i: (0, 0)),
            pl.BlockSpec((1, H), lambda i: (0, 0)),
            pl.BlockSpec((H, H), lambda i: (0, 0)),
        ],
        out_specs=[pl.BlockSpec((BR, H), lambda i: (i, 0)),
                   pl.BlockSpec((BR, 1), lambda i: (i, 0))],
        out_shape=[jax.ShapeDtypeStruct((NP, H), _F32),
                   jax.ShapeDtypeStruct((NP, 1), _F32)],
    )(s0, x_aug, wl, bl, wr)


def _tc_layer(s, cnt, h_prev, wl, bl, wr):
    def body(s_ref, c_ref, hp_ref, wl_ref, bl_ref, wr_ref, h_ref):
        mean = (s_ref[0] + s_ref[1]) / c_ref[...]
        h = (jnp.dot(mean, wl_ref[...], preferred_element_type=_F32, precision=_HIGH)
             + bl_ref[...]
             + jnp.dot(hp_ref[...], wr_ref[...], preferred_element_type=_F32, precision=_HIGH))
        h_ref[...] = jnp.maximum(h, 0.0)

    return pl.pallas_call(
        body,
        grid=(NB,),
        in_specs=[
            pl.BlockSpec((NC, BR, H), lambda i: (0, i, 0)),
            pl.BlockSpec((BR, 1), lambda i: (i, 0)),
            pl.BlockSpec((BR, H), lambda i: (i, 0)),
            pl.BlockSpec((H, H), lambda i: (0, 0)),
            pl.BlockSpec((1, H), lambda i: (0, 0)),
            pl.BlockSpec((H, H), lambda i: (0, 0)),
        ],
        out_specs=pl.BlockSpec((BR, H), lambda i: (i, 0)),
        out_shape=jax.ShapeDtypeStruct((NP, H), _F32),
    )(s, cnt, h_prev, wl, bl, wr)


def _tc_final(s, cnt, h_prev, wl, bl, wr, batch3, wlin, blin):
    def body(s_ref, c_ref, hp_ref, wl_ref, bl_ref, wr_ref, b_ref,
             wlin_ref, blin_ref, o_ref, pool_acc, cnt_acc):
        i = pl.program_id(0)

        @pl.when(i == 0)
        def _():
            pool_acc[...] = jnp.zeros((G, H), _F32)
            cnt_acc[...] = jnp.zeros((G, 1), _F32)

        mean = (s_ref[0] + s_ref[1]) / c_ref[...]
        h2 = (jnp.dot(mean, wl_ref[...], preferred_element_type=_F32, precision=_HIGH)
              + bl_ref[...]
              + jnp.dot(hp_ref[...], wr_ref[...], preferred_element_type=_F32, precision=_HIGH))
        b = b_ref[0, 0, :]
        oh_t = (lax.broadcasted_iota(jnp.int32, (G, BR), 0)
                == b[None, :]).astype(_F32)
        pool_acc[...] += jnp.dot(oh_t, h2, preferred_element_type=_F32, precision=_HIGH)
        cnt_acc[...] += jnp.sum(oh_t, axis=1, keepdims=True)

        @pl.when(i == NB - 1)
        def _():
            pooled = pool_acc[...] / jnp.maximum(cnt_acc[...], 1.0)
            z = jnp.dot(pooled, wlin_ref[...], preferred_element_type=_F32,
                        precision=_HIGH) + blin_ref[...]
            o_ref[...] = jax.nn.sigmoid(z)

    return pl.pallas_call(
        body,
        grid=(NB,),
        in_specs=[
            pl.BlockSpec((NC, BR, H), lambda i: (0, i, 0)),
            pl.BlockSpec((BR, 1), lambda i: (i, 0)),
            pl.BlockSpec((BR, H), lambda i: (i, 0)),
            pl.BlockSpec((H, H), lambda i: (0, 0)),
            pl.BlockSpec((1, H), lambda i: (0, 0)),
            pl.BlockSpec((H, H), lambda i: (0, 0)),
            pl.BlockSpec((1, 1, BR), lambda i: (i, 0, 0)),
            pl.BlockSpec((H, 1), lambda i: (0, 0)),
            pl.BlockSpec((1, 1), lambda i: (0, 0)),
        ],
        out_specs=pl.BlockSpec((G, 1), lambda i: (0, 0)),
        out_shape=jax.ShapeDtypeStruct((G, 1), _F32),
        scratch_shapes=[pltpu.VMEM((G, H), _F32), pltpu.VMEM((G, 1), _F32)],
    )(s, cnt, h_prev, wl, bl, wr, batch3, wlin, blin)


def kernel(x, edge_index, batch, Wl0, bl0, Wr0, Wl1, bl1, Wr1, Wl2, bl2, Wr2,
           Wlin, blin):
    pad = EPWP - EPW
    src4 = jnp.concatenate(
        [edge_index[0].reshape(NW, EPW),
         jnp.zeros((NW, pad), jnp.int32)], axis=1).reshape(NW, GCH, 1, CH)
    dst4 = jnp.concatenate(
        [edge_index[1].reshape(NW, EPW),
         jnp.full((NW, pad), NP - 1, jnp.int32)], axis=1).reshape(NW, GCH, 1, CH)
    x_aug = jnp.concatenate(
        [x, jnp.ones((N, 1), _F32), jnp.zeros((N, H - 6), _F32)], axis=1)
    x_aug = jnp.pad(x_aug, ((0, NP - N), (0, 0)))
    batch_p = jnp.pad(batch, (0, NP - N), constant_values=G)
    wl0p = jnp.concatenate([Wl0, jnp.zeros((H - 5, H), _F32)], axis=0)
    wr0p = jnp.concatenate([Wr0, jnp.zeros((H - 5, H), _F32)], axis=0)
    z128 = jnp.zeros((NPT, H), _F32)

    s0 = _sc_agg(x_aug, src4, dst4, z128, H)
    h0, cnt = _tc_layer0(s0, x_aug, wl0p, bl0.reshape(1, H), wr0p)
    s1 = _sc_agg(h0, src4, dst4, z128, H)
    h1 = _tc_layer(s1, cnt, h0, Wl1, bl1.reshape(1, H), Wr1)
    s2 = _sc_agg(h1, src4, dst4, z128, H)
    out = _tc_final(s2, cnt, h1, Wl2, bl2.reshape(1, H), Wr2,
                    batch_p.reshape(NB, 1, BR), Wlin, blin.reshape(1, 1))
    return out


# speedup vs baseline: 1.0001x; 1.0001x over previous
import jax
import jax.numpy as jnp
from jax import lax
from jax.experimental import pallas as pl
from jax.experimental.pallas import tpu as pltpu
from jax.experimental.pallas import tpu_sc as plsc

N = 10000
E = 320000
H = 128
G = 64

NP = 10240
NC = 2
NS = 16
NW = NC * NS
EPW = E // NW
CH = 128
GCH = 80
EPWP = GCH * CH
NPT = NP // NS
BR = 512
NB = NP // BR
LI = 4
KR = 2

_F32 = jnp.float32
_HIGH = lax.Precision.HIGHEST


def _sc_agg(h, src4, dst4, zrows, w):
    mesh = plsc.VectorSubcoreMesh(core_axis_name="c", subcore_axis_name="s")

    def body(h_hbm, src_hbm, dst_hbm, z_hbm, out_hbm, srcv, dstv, rows0,
             rows1, accum, g0, g1, d0, d1, d2, d3):
        rowsb = (rows0, rows1)
        gsems = (g0, g1)
        dsems = (d0, d1, d2, d3)
        cid = lax.axis_index("c")
        sid = lax.axis_index("s")
        wid = cid * NS + sid
        r0 = sid * NPT
        pltpu.sync_copy(z_hbm, accum.at[pl.ds(r0, NPT)])
        pltpu.sync_copy(src_hbm.at[wid], srcv)
        for j in range(LI):
            pltpu.async_copy(dst_hbm.at[wid, j], dstv.at[j], dsems[j])
        for k in range(KR):
            pltpu.async_copy(h_hbm.at[srcv.at[k, 0]], rowsb[k], gsems[k])
        plsc.subcore_barrier()

        def step(so, carry):
            for u in range(LI):
                g = so * LI + u
                k = u % KR
                pltpu.make_async_copy(h_hbm.at[srcv.at[u, 0]], rowsb[k],
                                      gsems[k]).wait()
                pltpu.make_async_copy(dst_hbm.at[wid, 0], dstv.at[u],
                                      dsems[u]).wait()
                pltpu.sync_copy(rowsb[k], accum.at[dstv.at[u, 0]], add=True)
                gn = jnp.minimum(g + LI, GCH - 1)
                pltpu.async_copy(dst_hbm.at[wid, gn], dstv.at[u], dsems[u])
                g2 = jnp.minimum(g + 2, GCH - 1)
                pltpu.async_copy(h_hbm.at[srcv.at[g2, 0]], rowsb[k], gsems[k])
            return carry

        lax.fori_loop(0, GCH // LI, step, 0)
        for k in range(KR):
            pltpu.make_async_copy(h_hbm.at[srcv.at[k, 0]], rowsb[k],
                                  gsems[k]).wait()
        for j in range(LI):
            pltpu.make_async_copy(dst_hbm.at[wid, 0], dstv.at[j], dsems[j]).wait()
        plsc.subcore_barrier()
        pltpu.sync_copy(accum.at[pl.ds(r0, NPT)],
                        out_hbm.at[cid, pl.ds(r0, NPT)])

    call = pl.kernel(
        body,
        out_type=jax.ShapeDtypeStruct((NC, NP, w), _F32),
        mesh=mesh,
        scratch_types=[
            pltpu.VMEM((GCH, 1, CH), jnp.int32),
            pltpu.VMEM((LI, 1, CH), jnp.int32),
            pltpu.VMEM((CH, w), _F32),
            pltpu.VMEM((CH, w), _F32),
            pltpu.VMEM_SHARED((NP, w), _F32),
            pltpu.SemaphoreType.DMA,
            pltpu.SemaphoreType.DMA,
            pltpu.SemaphoreType.DMA,
            pltpu.SemaphoreType.DMA,
            pltpu.SemaphoreType.DMA,
            pltpu.SemaphoreType.DMA,
        ],
    )
    return call(h, src4, dst4, zrows)


def _tc_layer0(s0, x_aug, wl, bl, wr):
    def body(s_ref, x_ref, wl_ref, bl_ref, wr_ref, h_ref, cnt_ref):
        s = s_ref[0] + s_ref[1]
        cnt = jnp.maximum(s[:, 5:6], 1.0)
        mean = s / cnt
        h = (jnp.dot(mean, wl_ref[...], preferred_element_type=_F32, precision=_HIGH)
             + bl_ref[...]
             + jnp.dot(x_ref[...], wr_ref[...], preferred_element_type=_F32, precision=_HIGH))
        h_ref[...] = jnp.maximum(h, 0.0)
        cnt_ref[...] = cnt

    return pl.pallas_call(
        body,
        grid=(NB,),
        in_specs=[
            pl.BlockSpec((NC, BR, H), lambda i: (0, i, 0)),
            pl.BlockSpec((BR, H), lambda i: (i, 0)),
            pl.BlockSpec((H, H), lambda i: (0, 0)),
            pl.BlockSpec((1, H), lambda i: (0, 0)),
            pl.BlockSpec((H, H), lambda i: (0, 0)),
        ],
        out_specs=[pl.BlockSpec((BR, H), lambda i: (i, 0)),
                   pl.BlockSpec((BR, 1), lambda i: (i, 0))],
        out_shape=[jax.ShapeDtypeStruct((NP, H), _F32),
                   jax.ShapeDtypeStruct((NP, 1), _F32)],
    )(s0, x_aug, wl, bl, wr)


def _tc_layer(s, cnt, h_prev, wl, bl, wr):
    def body(s_ref, c_ref, hp_ref, wl_ref, bl_ref, wr_ref, h_ref):
        mean = (s_ref[0] + s_ref[1]) / c_ref[...]
        h = (jnp.dot(mean, wl_ref[...], preferred_element_type=_F32, precision=_HIGH)
             + bl_ref[...]
             + jnp.dot(hp_ref[...], wr_ref[...], preferred_element_type=_F32, precision=_HIGH))
        h_ref[...] = jnp.maximum(h, 0.0)

    return pl.pallas_call(
        body,
        grid=(NB,),
        in_specs=[
            pl.BlockSpec((NC, BR, H), lambda i: (0, i, 0)),
            pl.BlockSpec((BR, 1), lambda i: (i, 0)),
            pl.BlockSpec((BR, H), lambda i: (i, 0)),
            pl.BlockSpec((H, H), lambda i: (0, 0)),
            pl.BlockSpec((1, H), lambda i: (0, 0)),
            pl.BlockSpec((H, H), lambda i: (0, 0)),
        ],
        out_specs=pl.BlockSpec((BR, H), lambda i: (i, 0)),
        out_shape=jax.ShapeDtypeStruct((NP, H), _F32),
    )(s, cnt, h_prev, wl, bl, wr)


def _tc_final(s, cnt, h_prev, wl, bl, wr, batch3, wlin, blin):
    def body(s_ref, c_ref, hp_ref, wl_ref, bl_ref, wr_ref, b_ref,
             wlin_ref, blin_ref, o_ref, pool_acc, cnt_acc):
        i = pl.program_id(0)

        @pl.when(i == 0)
        def _():
            pool_acc[...] = jnp.zeros((G, H), _F32)
            cnt_acc[...] = jnp.zeros((G, 1), _F32)

        mean = (s_ref[0] + s_ref[1]) / c_ref[...]
        h2 = (jnp.dot(mean, wl_ref[...], preferred_element_type=_F32, precision=_HIGH)
              + bl_ref[...]
              + jnp.dot(hp_ref[...], wr_ref[...], preferred_element_type=_F32, precision=_HIGH))
        b = b_ref[0, 0, :]
        oh_t = (lax.broadcasted_iota(jnp.int32, (G, BR), 0)
                == b[None, :]).astype(_F32)
        pool_acc[...] += jnp.dot(oh_t, h2, preferred_element_type=_F32, precision=_HIGH)
        cnt_acc[...] += jnp.sum(oh_t, axis=1, keepdims=True)

        @pl.when(i == NB - 1)
        def _():
            pooled = pool_acc[...] / jnp.maximum(cnt_acc[...], 1.0)
            z = jnp.dot(pooled, wlin_ref[...], preferred_element_type=_F32,
                        precision=_HIGH) + blin_ref[...]
            o_ref[...] = jax.nn.sigmoid(z)

    return pl.pallas_call(
        body,
        grid=(NB,),
        in_specs=[
            pl.BlockSpec((NC, BR, H), lambda i: (0, i, 0)),
            pl.BlockSpec((BR, 1), lambda i: (i, 0)),
            pl.BlockSpec((BR, H), lambda i: (i, 0)),
            pl.BlockSpec((H, H), lambda i: (0, 0)),
            pl.BlockSpec((1, H), lambda i: (0, 0)),
            pl.BlockSpec((H, H), lambda i: (0, 0)),
            pl.BlockSpec((1, 1, BR), lambda i: (i, 0, 0)),
            pl.BlockSpec((H, 1), lambda i: (0, 0)),
            pl.BlockSpec((1, 1), lambda i: (0, 0)),
        ],
        out_specs=pl.BlockSpec((G, 1), lambda i: (0, 0)),
        out_shape=jax.ShapeDtypeStruct((G, 1), _F32),
        scratch_shapes=[pltpu.VMEM((G, H), _F32), pltpu.VMEM((G, 1), _F32)],
    )(s, cnt, h_prev, wl, bl, wr, batch3, wlin, blin)


def kernel(x, edge_index, batch, Wl0, bl0, Wr0, Wl1, bl1, Wr1, Wl2, bl2, Wr2,
           Wlin, blin):
    pad = EPWP - EPW
    src4 = jnp.concatenate(
        [edge_index[0].reshape(NW, EPW),
         jnp.zeros((NW, pad), jnp.int32)], axis=1).reshape(NW, GCH, 1, CH)
    dump = N + jnp.arange(NW, dtype=jnp.int32)[:, None]
    dst4 = jnp.concatenate(
        [edge_index[1].reshape(NW, EPW),
         jnp.broadcast_to(dump, (NW, pad))], axis=1).reshape(NW, GCH, 1, CH)
    x_aug = jnp.concatenate(
        [x, jnp.ones((N, 1), _F32), jnp.zeros((N, H - 6), _F32)], axis=1)
    x_aug = jnp.pad(x_aug, ((0, NP - N), (0, 0)))
    batch_p = jnp.pad(batch, (0, NP - N), constant_values=G)
    wl0p = jnp.concatenate([Wl0, jnp.zeros((H - 5, H), _F32)], axis=0)
    wr0p = jnp.concatenate([Wr0, jnp.zeros((H - 5, H), _F32)], axis=0)
    z128 = jnp.zeros((NPT, H), _F32)

    s0 = _sc_agg(x_aug, src4, dst4, z128, H)
    h0, cnt = _tc_layer0(s0, x_aug, wl0p, bl0.reshape(1, H), wr0p)
    s1 = _sc_agg(h0, src4, dst4, z128, H)
    h1 = _tc_layer(s1, cnt, h0, Wl1, bl1.reshape(1, H), Wr1)
    s2 = _sc_agg(h1, src4, dst4, z128, H)
    out = _tc_final(s2, cnt, h1, Wl2, bl2.reshape(1, H), Wr2,
                    batch_p.reshape(NB, 1, BR), Wlin, blin.reshape(1, 1))
    return out


# speedup vs baseline: 5.0016x; 5.0011x over previous
import jax
import jax.numpy as jnp
from jax import lax
from jax.experimental import pallas as pl
from jax.experimental.pallas import tpu as pltpu
from jax.experimental.pallas import tpu_sc as plsc

N = 10000
E = 320000
H = 128
G = 64

NP = 10240
NC = 2
NS = 16
NW = NC * NS
EPW = E // NW
CH = 125
GCH = 80
EPWP = GCH * CH
NPT = NP // NS
BR = 512
NB = NP // BR
LI = 4
KR = 2

_F32 = jnp.float32
_HIGH = lax.Precision.HIGHEST


def _sc_agg(h, src4, dst4, zrows, w):
    mesh = plsc.VectorSubcoreMesh(core_axis_name="c", subcore_axis_name="s")

    def body(h_hbm, src_hbm, dst_hbm, z_hbm, out_hbm, srcv, dstv, rows0,
             rows1, accum, g0, g1, d0, d1, d2, d3):
        rowsb = (rows0, rows1)
        gsems = (g0, g1)
        dsems = (d0, d1, d2, d3)
        cid = lax.axis_index("c")
        sid = lax.axis_index("s")
        wid = cid * NS + sid
        r0 = sid * NPT
        pltpu.sync_copy(z_hbm, accum.at[pl.ds(r0, NPT)])
        pltpu.sync_copy(src_hbm.at[wid], srcv)
        for j in range(LI):
            pltpu.async_copy(dst_hbm.at[wid, j], dstv.at[j], dsems[j])
        for k in range(KR):
            pltpu.async_copy(h_hbm.at[srcv.at[k, 0]], rowsb[k], gsems[k])
        plsc.subcore_barrier()

        def step(so, carry):
            for u in range(LI):
                g = so * LI + u
                k = u % KR
                pltpu.make_async_copy(h_hbm.at[srcv.at[u, 0]], rowsb[k],
                                      gsems[k]).wait()
                pltpu.make_async_copy(dst_hbm.at[wid, 0], dstv.at[u],
                                      dsems[u]).wait()
                pltpu.sync_copy(rowsb[k], accum.at[dstv.at[u, 0]], add=True)
                gn = jnp.minimum(g + LI, GCH - 1)
                pltpu.async_copy(dst_hbm.at[wid, gn], dstv.at[u], dsems[u])
                g2 = jnp.minimum(g + 2, GCH - 1)
                pltpu.async_copy(h_hbm.at[srcv.at[g2, 0]], rowsb[k], gsems[k])
            return carry

        lax.fori_loop(0, GCH // LI, step, 0)
        for k in range(KR):
            pltpu.make_async_copy(h_hbm.at[srcv.at[k, 0]], rowsb[k],
                                  gsems[k]).wait()
        for j in range(LI):
            pltpu.make_async_copy(dst_hbm.at[wid, 0], dstv.at[j], dsems[j]).wait()
        plsc.subcore_barrier()
        pltpu.sync_copy(accum.at[pl.ds(r0, NPT)],
                        out_hbm.at[cid, pl.ds(r0, NPT)])

    call = pl.kernel(
        body,
        out_type=jax.ShapeDtypeStruct((NC, NP, w), _F32),
        mesh=mesh,
        scratch_types=[
            pltpu.VMEM((GCH, 1, CH), jnp.int32),
            pltpu.VMEM((LI, 1, CH), jnp.int32),
            pltpu.VMEM((CH, w), _F32),
            pltpu.VMEM((CH, w), _F32),
            pltpu.VMEM_SHARED((NP, w), _F32),
            pltpu.SemaphoreType.DMA,
            pltpu.SemaphoreType.DMA,
            pltpu.SemaphoreType.DMA,
            pltpu.SemaphoreType.DMA,
            pltpu.SemaphoreType.DMA,
            pltpu.SemaphoreType.DMA,
        ],
    )
    return call(h, src4, dst4, zrows)


def _tc_layer0(s0, x_aug, wl, bl, wr):
    def body(s_ref, x_ref, wl_ref, bl_ref, wr_ref, h_ref, cnt_ref):
        s = s_ref[0] + s_ref[1]
        cnt = jnp.maximum(s[:, 5:6], 1.0)
        mean = s / cnt
        h = (jnp.dot(mean, wl_ref[...], preferred_element_type=_F32, precision=_HIGH)
             + bl_ref[...]
             + jnp.dot(x_ref[...], wr_ref[...], preferred_element_type=_F32, precision=_HIGH))
        h_ref[...] = jnp.maximum(h, 0.0)
        cnt_ref[...] = cnt

    return pl.pallas_call(
        body,
        grid=(NB,),
        in_specs=[
            pl.BlockSpec((NC, BR, H), lambda i: (0, i, 0)),
            pl.BlockSpec((BR, H), lambda i: (i, 0)),
            pl.BlockSpec((H, H), lambda i: (0, 0)),
            pl.BlockSpec((1, H), lambda i: (0, 0)),
            pl.BlockSpec((H, H), lambda i: (0, 0)),
        ],
        out_specs=[pl.BlockSpec((BR, H), lambda i: (i, 0)),
                   pl.BlockSpec((BR, 1), lambda i: (i, 0))],
        out_shape=[jax.ShapeDtypeStruct((NP, H), _F32),
                   jax.ShapeDtypeStruct((NP, 1), _F32)],
    )(s0, x_aug, wl, bl, wr)


def _tc_layer(s, cnt, h_prev, wl, bl, wr):
    def body(s_ref, c_ref, hp_ref, wl_ref, bl_ref, wr_ref, h_ref):
        mean = (s_ref[0] + s_ref[1]) / c_ref[...]
        h = (jnp.dot(mean, wl_ref[...], preferred_element_type=_F32, precision=_HIGH)
             + bl_ref[...]
             + jnp.dot(hp_ref[...], wr_ref[...], preferred_element_type=_F32, precision=_HIGH))
        h_ref[...] = jnp.maximum(h, 0.0)

    return pl.pallas_call(
        body,
        grid=(NB,),
        in_specs=[
            pl.BlockSpec((NC, BR, H), lambda i: (0, i, 0)),
            pl.BlockSpec((BR, 1), lambda i: (i, 0)),
            pl.BlockSpec((BR, H), lambda i: (i, 0)),
            pl.BlockSpec((H, H), lambda i: (0, 0)),
            pl.BlockSpec((1, H), lambda i: (0, 0)),
            pl.BlockSpec((H, H), lambda i: (0, 0)),
        ],
        out_specs=pl.BlockSpec((BR, H), lambda i: (i, 0)),
        out_shape=jax.ShapeDtypeStruct((NP, H), _F32),
    )(s, cnt, h_prev, wl, bl, wr)


def _tc_final(s, cnt, h_prev, wl, bl, wr, batch3, wlin, blin):
    def body(s_ref, c_ref, hp_ref, wl_ref, bl_ref, wr_ref, b_ref,
             wlin_ref, blin_ref, o_ref, pool_acc, cnt_acc):
        i = pl.program_id(0)

        @pl.when(i == 0)
        def _():
            pool_acc[...] = jnp.zeros((G, H), _F32)
            cnt_acc[...] = jnp.zeros((G, 1), _F32)

        mean = (s_ref[0] + s_ref[1]) / c_ref[...]
        h2 = (jnp.dot(mean, wl_ref[...], preferred_element_type=_F32, precision=_HIGH)
              + bl_ref[...]
              + jnp.dot(hp_ref[...], wr_ref[...], preferred_element_type=_F32, precision=_HIGH))
        b = b_ref[0, 0, :]
        oh_t = (lax.broadcasted_iota(jnp.int32, (G, BR), 0)
                == b[None, :]).astype(_F32)
        pool_acc[...] += jnp.dot(oh_t, h2, preferred_element_type=_F32, precision=_HIGH)
        cnt_acc[...] += jnp.sum(oh_t, axis=1, keepdims=True)

        @pl.when(i == NB - 1)
        def _():
            pooled = pool_acc[...] / jnp.maximum(cnt_acc[...], 1.0)
            z = jnp.dot(pooled, wlin_ref[...], preferred_element_type=_F32,
                        precision=_HIGH) + blin_ref[...]
            o_ref[...] = jax.nn.sigmoid(z)

    return pl.pallas_call(
        body,
        grid=(NB,),
        in_specs=[
            pl.BlockSpec((NC, BR, H), lambda i: (0, i, 0)),
            pl.BlockSpec((BR, 1), lambda i: (i, 0)),
            pl.BlockSpec((BR, H), lambda i: (i, 0)),
            pl.BlockSpec((H, H), lambda i: (0, 0)),
            pl.BlockSpec((1, H), lambda i: (0, 0)),
            pl.BlockSpec((H, H), lambda i: (0, 0)),
            pl.BlockSpec((1, 1, BR), lambda i: (i, 0, 0)),
            pl.BlockSpec((H, 1), lambda i: (0, 0)),
            pl.BlockSpec((1, 1), lambda i: (0, 0)),
        ],
        out_specs=pl.BlockSpec((G, 1), lambda i: (0, 0)),
        out_shape=jax.ShapeDtypeStruct((G, 1), _F32),
        scratch_shapes=[pltpu.VMEM((G, H), _F32), pltpu.VMEM((G, 1), _F32)],
    )(s, cnt, h_prev, wl, bl, wr, batch3, wlin, blin)


def kernel(x, edge_index, batch, Wl0, bl0, Wr0, Wl1, bl1, Wr1, Wl2, bl2, Wr2,
           Wlin, blin):
    pad = EPWP - EPW
    src4 = jnp.concatenate(
        [edge_index[0].reshape(NW, EPW),
         jnp.zeros((NW, pad), jnp.int32)], axis=1).reshape(NW, GCH, 1, CH)
    dump = N + jnp.arange(NW, dtype=jnp.int32)[:, None]
    dst4 = jnp.concatenate(
        [edge_index[1].reshape(NW, EPW),
         jnp.broadcast_to(dump, (NW, pad))], axis=1).reshape(NW, GCH, 1, CH)
    x_aug = jnp.concatenate(
        [x, jnp.ones((N, 1), _F32), jnp.zeros((N, H - 6), _F32)], axis=1)
    x_aug = jnp.pad(x_aug, ((0, NP - N), (0, 0)))
    batch_p = jnp.pad(batch, (0, NP - N), constant_values=G)
    wl0p = jnp.concatenate([Wl0, jnp.zeros((H - 5, H), _F32)], axis=0)
    wr0p = jnp.concatenate([Wr0, jnp.zeros((H - 5, H), _F32)], axis=0)
    z128 = jnp.zeros((NPT, H), _F32)

    s0 = _sc_agg(x_aug, src4, dst4, z128, H)
    h0, cnt = _tc_layer0(s0, x_aug, wl0p, bl0.reshape(1, H), wr0p)
    s1 = _sc_agg(h0, src4, dst4, z128, H)
    h1 = _tc_layer(s1, cnt, h0, Wl1, bl1.reshape(1, H), Wr1)
    s2 = _sc_agg(h1, src4, dst4, z128, H)
    out = _tc_final(s2, cnt, h1, Wl2, bl2.reshape(1, H), Wr2,
                    batch_p.reshape(NB, 1, BR), Wlin, blin.reshape(1, 1))
    return out


# speedup vs baseline: 5.3853x; 1.0767x over previous
import jax
import jax.numpy as jnp
from jax import lax
from jax.experimental import pallas as pl
from jax.experimental.pallas import tpu as pltpu
from jax.experimental.pallas import tpu_sc as plsc

N = 10000
E = 320000
H = 128
G = 64

NP = 10240
NC = 2
NS = 16
NW = NC * NS
EPW = E // NW
CH = 125
GCH = 80
EPWP = GCH * CH
NPT = NP // NS
BR = 1024
NB = NP // BR
LI = 4
KR = 2

_F32 = jnp.float32
_HIGH = lax.Precision.DEFAULT


def _sc_agg(h, src4, dst4, zrows, w):
    mesh = plsc.VectorSubcoreMesh(core_axis_name="c", subcore_axis_name="s")

    def body(h_hbm, src_hbm, dst_hbm, z_hbm, out_hbm, srcv, dstv, rows0,
             rows1, accum, g0, g1, d0, d1, d2, d3):
        rowsb = (rows0, rows1)
        gsems = (g0, g1)
        dsems = (d0, d1, d2, d3)
        cid = lax.axis_index("c")
        sid = lax.axis_index("s")
        wid = cid * NS + sid
        r0 = sid * NPT
        pltpu.sync_copy(z_hbm, accum.at[pl.ds(r0, NPT)])
        pltpu.sync_copy(src_hbm.at[wid], srcv)
        for j in range(LI):
            pltpu.async_copy(dst_hbm.at[wid, j], dstv.at[j], dsems[j])
        for k in range(KR):
            pltpu.async_copy(h_hbm.at[srcv.at[k, 0]], rowsb[k], gsems[k])
        plsc.subcore_barrier()

        def step(so, carry):
            for u in range(LI):
                g = so * LI + u
                k = u % KR
                pltpu.make_async_copy(h_hbm.at[srcv.at[u, 0]], rowsb[k],
                                      gsems[k]).wait()
                pltpu.make_async_copy(dst_hbm.at[wid, 0], dstv.at[u],
                                      dsems[u]).wait()
                pltpu.sync_copy(rowsb[k], accum.at[dstv.at[u, 0]], add=True)
                gn = jnp.minimum(g + LI, GCH - 1)
                pltpu.async_copy(dst_hbm.at[wid, gn], dstv.at[u], dsems[u])
                g2 = jnp.minimum(g + 2, GCH - 1)
                pltpu.async_copy(h_hbm.at[srcv.at[g2, 0]], rowsb[k], gsems[k])
            return carry

        lax.fori_loop(0, GCH // LI, step, 0)
        for k in range(KR):
            pltpu.make_async_copy(h_hbm.at[srcv.at[k, 0]], rowsb[k],
                                  gsems[k]).wait()
        for j in range(LI):
            pltpu.make_async_copy(dst_hbm.at[wid, 0], dstv.at[j], dsems[j]).wait()
        plsc.subcore_barrier()
        pltpu.sync_copy(accum.at[pl.ds(r0, NPT)],
                        out_hbm.at[cid, pl.ds(r0, NPT)])

    call = pl.kernel(
        body,
        out_type=jax.ShapeDtypeStruct((NC, NP, w), _F32),
        mesh=mesh,
        scratch_types=[
            pltpu.VMEM((GCH, 1, CH), jnp.int32),
            pltpu.VMEM((LI, 1, CH), jnp.int32),
            pltpu.VMEM((CH, w), _F32),
            pltpu.VMEM((CH, w), _F32),
            pltpu.VMEM_SHARED((NP, w), _F32),
            pltpu.SemaphoreType.DMA,
            pltpu.SemaphoreType.DMA,
            pltpu.SemaphoreType.DMA,
            pltpu.SemaphoreType.DMA,
            pltpu.SemaphoreType.DMA,
            pltpu.SemaphoreType.DMA,
        ],
    )
    return call(h, src4, dst4, zrows)


def _tc_layer0(s0, x_aug, wl, bl, wr):
    def body(s_ref, x_ref, wl_ref, bl_ref, wr_ref, h_ref, cnt_ref):
        s = s_ref[0] + s_ref[1]
        cnt = jnp.maximum(s[:, 5:6], 1.0)
        mean = s / cnt
        h = (jnp.dot(mean, wl_ref[...], preferred_element_type=_F32, precision=_HIGH)
             + bl_ref[...]
             + jnp.dot(x_ref[...], wr_ref[...], preferred_element_type=_F32, precision=_HIGH))
        h_ref[...] = jnp.maximum(h, 0.0)
        cnt_ref[...] = cnt

    return pl.pallas_call(
        body,
        grid=(NB,),
        in_specs=[
            pl.BlockSpec((NC, BR, H), lambda i: (0, i, 0)),
            pl.BlockSpec((BR, H), lambda i: (i, 0)),
            pl.BlockSpec((H, H), lambda i: (0, 0)),
            pl.BlockSpec((1, H), lambda i: (0, 0)),
            pl.BlockSpec((H, H), lambda i: (0, 0)),
        ],
        out_specs=[pl.BlockSpec((BR, H), lambda i: (i, 0)),
                   pl.BlockSpec((BR, 1), lambda i: (i, 0))],
        out_shape=[jax.ShapeDtypeStruct((NP, H), _F32),
                   jax.ShapeDtypeStruct((NP, 1), _F32)],
    )(s0, x_aug, wl, bl, wr)


def _tc_layer(s, cnt, h_prev, wl, bl, wr):
    def body(s_ref, c_ref, hp_ref, wl_ref, bl_ref, wr_ref, h_ref):
        mean = (s_ref[0] + s_ref[1]) / c_ref[...]
        h = (jnp.dot(mean, wl_ref[...], preferred_element_type=_F32, precision=_HIGH)
             + bl_ref[...]
             + jnp.dot(hp_ref[...], wr_ref[...], preferred_element_type=_F32, precision=_HIGH))
        h_ref[...] = jnp.maximum(h, 0.0)

    return pl.pallas_call(
        body,
        grid=(NB,),
        in_specs=[
            pl.BlockSpec((NC, BR, H), lambda i: (0, i, 0)),
            pl.BlockSpec((BR, 1), lambda i: (i, 0)),
            pl.BlockSpec((BR, H), lambda i: (i, 0)),
            pl.BlockSpec((H, H), lambda i: (0, 0)),
            pl.BlockSpec((1, H), lambda i: (0, 0)),
            pl.BlockSpec((H, H), lambda i: (0, 0)),
        ],
        out_specs=pl.BlockSpec((BR, H), lambda i: (i, 0)),
        out_shape=jax.ShapeDtypeStruct((NP, H), _F32),
    )(s, cnt, h_prev, wl, bl, wr)


def _tc_final(s, cnt, h_prev, wl, bl, wr, batch3, wlin, blin):
    def body(s_ref, c_ref, hp_ref, wl_ref, bl_ref, wr_ref, b_ref,
             wlin_ref, blin_ref, o_ref, pool_acc, cnt_acc):
        i = pl.program_id(0)

        @pl.when(i == 0)
        def _():
            pool_acc[...] = jnp.zeros((G, H), _F32)
            cnt_acc[...] = jnp.zeros((G, 1), _F32)

        mean = (s_ref[0] + s_ref[1]) / c_ref[...]
        h2 = (jnp.dot(mean, wl_ref[...], preferred_element_type=_F32, precision=_HIGH)
              + bl_ref[...]
              + jnp.dot(hp_ref[...], wr_ref[...], preferred_element_type=_F32, precision=_HIGH))
        b = b_ref[0, 0, :]
        oh_t = (lax.broadcasted_iota(jnp.int32, (G, BR), 0)
                == b[None, :]).astype(_F32)
        pool_acc[...] += jnp.dot(oh_t, h2, preferred_element_type=_F32, precision=_HIGH)
        cnt_acc[...] += jnp.sum(oh_t, axis=1, keepdims=True)

        @pl.when(i == NB - 1)
        def _():
            pooled = pool_acc[...] / jnp.maximum(cnt_acc[...], 1.0)
            z = jnp.dot(pooled, wlin_ref[...], preferred_element_type=_F32,
                        precision=_HIGH) + blin_ref[...]
            o_ref[...] = jax.nn.sigmoid(z)

    return pl.pallas_call(
        body,
        grid=(NB,),
        in_specs=[
            pl.BlockSpec((NC, BR, H), lambda i: (0, i, 0)),
            pl.BlockSpec((BR, 1), lambda i: (i, 0)),
            pl.BlockSpec((BR, H), lambda i: (i, 0)),
            pl.BlockSpec((H, H), lambda i: (0, 0)),
            pl.BlockSpec((1, H), lambda i: (0, 0)),
            pl.BlockSpec((H, H), lambda i: (0, 0)),
            pl.BlockSpec((1, 1, BR), lambda i: (i, 0, 0)),
            pl.BlockSpec((H, 1), lambda i: (0, 0)),
            pl.BlockSpec((1, 1), lambda i: (0, 0)),
        ],
        out_specs=pl.BlockSpec((G, 1), lambda i: (0, 0)),
        out_shape=jax.ShapeDtypeStruct((G, 1), _F32),
        scratch_shapes=[pltpu.VMEM((G, H), _F32), pltpu.VMEM((G, 1), _F32)],
    )(s, cnt, h_prev, wl, bl, wr, batch3, wlin, blin)


def kernel(x, edge_index, batch, Wl0, bl0, Wr0, Wl1, bl1, Wr1, Wl2, bl2, Wr2,
           Wlin, blin):
    pad = EPWP - EPW
    src4 = jnp.concatenate(
        [edge_index[0].reshape(NW, EPW),
         jnp.zeros((NW, pad), jnp.int32)], axis=1).reshape(NW, GCH, 1, CH)
    dump = N + jnp.arange(NW, dtype=jnp.int32)[:, None]
    dst4 = jnp.concatenate(
        [edge_index[1].reshape(NW, EPW),
         jnp.broadcast_to(dump, (NW, pad))], axis=1).reshape(NW, GCH, 1, CH)
    x_aug = jnp.concatenate(
        [x, jnp.ones((N, 1), _F32), jnp.zeros((N, H - 6), _F32)], axis=1)
    x_aug = jnp.pad(x_aug, ((0, NP - N), (0, 0)))
    batch_p = jnp.pad(batch, (0, NP - N), constant_values=G)
    wl0p = jnp.concatenate([Wl0, jnp.zeros((H - 5, H), _F32)], axis=0)
    wr0p = jnp.concatenate([Wr0, jnp.zeros((H - 5, H), _F32)], axis=0)
    z128 = jnp.zeros((NPT, H), _F32)

    s0 = _sc_agg(x_aug, src4, dst4, z128, H)
    h0, cnt = _tc_layer0(s0, x_aug, wl0p, bl0.reshape(1, H), wr0p)
    s1 = _sc_agg(h0, src4, dst4, z128, H)
    h1 = _tc_layer(s1, cnt, h0, Wl1, bl1.reshape(1, H), Wr1)
    s2 = _sc_agg(h1, src4, dst4, z128, H)
    out = _tc_final(s2, cnt, h1, Wl2, bl2.reshape(1, H), Wr2,
                    batch_p.reshape(NB, 1, BR), Wlin, blin.reshape(1, 1))
    return out
